# baseline (speedup 1.0000x reference)
import functools
import math

import jax
import jax.numpy as jnp
from jax import lax
from jax.experimental import pallas as pl
from jax.experimental.pallas import tpu as pltpu
from jax.experimental.pallas import tpu_sc as plsc

D_MODEL = 64
SCALE = math.sqrt(D_MODEL)

NC, NS, LANES = 2, 16, 16
NW = NC * NS

NB = 4


def _make_lookup(R, S, V):
    assert R % (NW * NB) == 0 and S == 200
    r_per_w = R // NW
    n_stages = r_per_w // NB
    S1, S2 = 128, S - 128
    H = S // 2
    mesh = plsc.VectorSubcoreMesh(core_axis_name="c", subcore_axis_name="s")

    @functools.partial(
        pl.kernel,
        out_type=jax.ShapeDtypeStruct((R * H, 2 * D_MODEL), jnp.float32),
        mesh=mesh,
        scratch_types=[
            pltpu.VMEM((NB, S1), jnp.int32),
            pltpu.VMEM((NB, S1), jnp.int32),
            pltpu.VMEM((NB, S, D_MODEL), jnp.float32),
            pltpu.VMEM((NB, H, 2 * D_MODEL), jnp.float32),
        ]
        + [pltpu.SemaphoreType.DMA] * (2 * NB),
        compiler_params=pltpu.CompilerParams(use_tc_tiling_on_sc=False),
    )
    def lookup(table_hbm, x1_hbm, x2_hbm, out_hbm, i1_v, i2_v, raw_v, pad_v, *sems):
        gsem, wsem = sems[:NB], sems[NB:]
        wid = lax.axis_index("s") * NC + lax.axis_index("c")
        row0 = wid * r_per_w

        def stage_body(ci, _):
            xbase = pl.multiple_of(row0 + ci * NB, NB)
            pltpu.sync_copy(x1_hbm.at[pl.ds(xbase, NB)], i1_v)
            pltpu.sync_copy(x2_hbm.at[pl.ds(xbase, NB)], i2_v)

            cps = [
                [
                    pltpu.async_copy(
                        table_hbm.at[i1_v.at[b]],
                        raw_v.at[b].at[pl.ds(0, S1)],
                        gsem[b],
                    ),
                    pltpu.async_copy(
                        table_hbm.at[i2_v.at[b].at[pl.ds(0, S2)]],
                        raw_v.at[b].at[pl.ds(S1, S2)],
                        gsem[b],
                    ),
                ]
                for b in range(NB)
            ]

            for b in range(NB):
                for cp in cps[b]:
                    cp.wait()

                @pl.when(ci > 0)
                def _():
                    pltpu.make_async_copy(
                        pad_v.at[b], out_hbm.at[pl.ds(0, H)], wsem[b]
                    ).wait()

                def scale_rows(q, _):
                    for i in range(2):
                        for v in range(2):
                            for c in range(D_MODEL // LANES):
                                pad_v[b, 2 * q + i, pl.ds(v * D_MODEL + c * LANES, LANES)] = (
                                    raw_v[b, 2 * (2 * q + i) + v, pl.ds(c * LANES, LANES)]
                                    * SCALE
                                )
                    return ()

                lax.fori_loop(0, H // 2, scale_rows, ())
                pltpu.async_copy(
                    pad_v.at[b],
                    out_hbm.at[pl.ds((xbase + b) * H, H)],
                    wsem[b],
                )
            return ()

        lax.fori_loop(0, n_stages, stage_body, ())
        for b in range(NB):
            pltpu.make_async_copy(
                pad_v.at[b], out_hbm.at[pl.ds(0, H)], wsem[b]
            ).wait()

    return lookup


def kernel(x, table):
    R, S = x.shape
    V = table.shape[0]
    xi = x.astype(jnp.int32)
    x1 = xi[:, :128]
    x2 = jnp.pad(xi[:, 128:], ((0, 0), (0, 256 - S)))
    out = _make_lookup(R, S, V)(table, x1, x2)
    return out.reshape(R, S, D_MODEL)
